# baseline (speedup 1.0000x reference)
import functools

import jax
import jax.numpy as jnp
from jax import lax
from jax.experimental import pallas as pl
from jax.experimental.pallas import tpu as pltpu
from jax.experimental.pallas import tpu_sc as plsc

BETA = 0.1
NC = 2
NS = 16
LANES = 16

_F32 = jnp.float32
_I32 = jnp.int32

_R = 1024


def _make_sc_deg(N, E, NT):
    C = 80
    W32 = NC * NS
    ep = E // W32
    nchunks = ep // C
    npad = (NT * 128) // NS
    mesh = plsc.VectorSubcoreMesh(
        core_axis_name="c", subcore_axis_name="s", num_cores=NC,
        num_subcores=NS)

    @functools.partial(
        pl.kernel, mesh=mesh,
        compiler_params=pltpu.CompilerParams(use_tc_tiling_on_sc=False, needs_layout_passes=False),
        out_type=jax.ShapeDtypeStruct((NC, NT, 128), _F32),
        scratch_types=[
            pltpu.VMEM((C,), _I32),
            pltpu.VMEM((C,), _F32),
            pltpu.VMEM((1024,), _F32),
            pltpu.VMEM((8, 128), _F32),
            pltpu.VMEM_SHARED((NT * 128,), _F32),
        ],
    )
    def deg_kernel(dst_hbm, out_hbm, dst_row, ones, v1d, v2d, cnt_sp):
        cid = lax.axis_index("c")
        sid = lax.axis_index("s")
        z16 = jnp.zeros((LANES,), _F32)
        o16 = jnp.ones((LANES,), _F32)

        def _fill(i, _):
            v1d[pl.ds(i * LANES, LANES)] = z16
            return ()
        lax.fori_loop(0, 1024 // LANES, _fill, ())
        for i in range(C // LANES):
            ones[pl.ds(i * LANES, LANES)] = o16
        pltpu.sync_copy(v1d.at[pl.ds(0, npad)],
                        cnt_sp.at[pl.ds(sid * npad, npad)])
        plsc.subcore_barrier()

        base = (sid * NC + cid) * ep

        def _chunk(c, _):
            pltpu.sync_copy(dst_hbm.at[pl.ds(base + c * C, C)], dst_row)
            pltpu.sync_copy(ones, cnt_sp.at[dst_row], add=True)
            return ()
        lax.fori_loop(0, nchunks, _chunk, ())
        plsc.subcore_barrier()

        @pl.when(sid < NT // 8)
        def _():
            pltpu.sync_copy(cnt_sp.at[pl.ds(sid * 1024, 1024)], v1d)
            for r in range(8):
                for c in range(128 // LANES):
                    v2d[r, pl.ds(c * LANES, LANES)] = (
                        v1d[pl.ds((r * 8 + c) * LANES, LANES)])
            pltpu.sync_copy(v2d, out_hbm.at[cid, pl.ds(sid * 8, 8)])

    return deg_kernel


def _make_sc_msg(N, D, E, NT):
    C = 80
    ep = E // NS
    nchunks = ep // C
    nquads = (nchunks + 3) // 4
    nio = 10
    rows_per_io = N // nio
    zr = 40
    mesh = plsc.VectorSubcoreMesh(
        core_axis_name="c", subcore_axis_name="s", num_cores=NC,
        num_subcores=NS)

    @functools.partial(
        pl.kernel, mesh=mesh,
        compiler_params=pltpu.CompilerParams(use_tc_tiling_on_sc=False, needs_layout_passes=False),
        out_type=jax.ShapeDtypeStruct((NC * N, D), _F32),
        scratch_types=[
            pltpu.VMEM((2, 4, C), _I32),
            pltpu.VMEM((2, 4, C), _I32),
            pltpu.VMEM((N, ), _F32),
            pltpu.VMEM((N, ), _F32),
            pltpu.VMEM((NT, 128), _F32),
            pltpu.VMEM((4, C, D), _F32),
            pltpu.VMEM((zr, D), _F32),
            pltpu.VMEM_SHARED((N, D), _F32),
            pltpu.SemaphoreType.DMA,
            pltpu.SemaphoreType.DMA,
            pltpu.SemaphoreType.DMA,
            pltpu.SemaphoreType.DMA,
            pltpu.SemaphoreType.DMA,
            pltpu.SemaphoreType.DMA,
            pltpu.SemaphoreType.DMA,
            pltpu.SemaphoreType.DMA,
            pltpu.SemaphoreType.DMA,
        ],
    )
    def msg_kernel(feat_hbm, gdgs_hbm, src_hbm, dst_hbm, out_hbm,
                   si, di, gd_t, gs_t, stage2d, rows, zbuf, agg_sp,
                   isem, g0, g1, g2, g3, s0, s1, s2, s3):
        cid = lax.axis_index("c")
        sid = lax.axis_index("s")
        off = cid * N
        gsems = (g0, g1, g2, g3)
        ssems = (s0, s1, s2, s3)

        def _flatten(t1d):
            def _row(r, _):
                for c in range(128 // LANES):
                    t1d[pl.ds(r * 128 + c * LANES, LANES)] = (
                        stage2d[r, pl.ds(c * LANES, LANES)])
                return ()
            lax.fori_loop(0, N // 128, _row, ())
            r = N // 128
            for c in range((N - (N // 128) * 128) // LANES):
                t1d[pl.ds(r * 128 + c * LANES, LANES)] = (
                    stage2d[r, pl.ds(c * LANES, LANES)])

        pltpu.sync_copy(gdgs_hbm.at[2 * cid], stage2d)
        _flatten(gd_t)
        pltpu.sync_copy(gdgs_hbm.at[2 * cid + 1], stage2d)
        _flatten(gs_t)

        z16 = jnp.zeros((LANES,), _F32)

        def _fill(r, _):
            for j in range(D // LANES):
                zbuf[r, pl.ds(j * LANES, LANES)] = z16
            return ()
        lax.fori_loop(0, zr, _fill, ())

        @pl.when(sid < nio)
        def _():
            def _zero(q, _):
                pltpu.sync_copy(
                    zbuf, agg_sp.at[pl.ds(sid * rows_per_io + q * zr, zr)])
                return ()
            lax.fori_loop(0, rows_per_io // zr, _zero, ())
        plsc.subcore_barrier()

        base_row = sid * nchunks

        def _fire_idx(quad_idx, parity):
            row = base_row + 4 * quad_idx
            pltpu.async_copy(src_hbm.at[pl.ds(row, 4)], si.at[parity], isem)
            pltpu.async_copy(dst_hbm.at[pl.ds(row, 4)], di.at[parity], isem)

        def _wait_idx():
            pltpu.make_async_copy(
                src_hbm.at[pl.ds(0, 4)], si.at[0], isem).wait()
            pltpu.make_async_copy(
                dst_hbm.at[pl.ds(0, 4)], di.at[0], isem).wait()

        def _wait_scat(u):
            pltpu.make_async_copy(
                rows.at[u], agg_sp.at[pl.ds(0, C)], ssems[u]).wait()

        _fire_idx(0, 0)

        def _quad(Q, _):
            pp = Q & 1
            qq = 1 - pp
            _wait_idx()
            for u in range(4):
                for g in range(C // LANES):
                    sl = pl.ds(g * LANES, LANES)
                    si[pp, u, sl] = si[pp, u, sl] + off
            @pl.when(Q > 0)
            def _():
                for u in range(4):
                    _wait_scat(u)
            @pl.when(Q + 1 < nquads)
            def _():
                _fire_idx(Q + 1, qq)
            for u in range(4):
                @pl.when(4 * Q + u < nchunks)
                def _(u=u):
                    pltpu.async_copy(
                        feat_hbm.at[si.at[pp, u]], rows.at[u], gsems[u])
            for u in range(4):
                @pl.when(4 * Q + u < nchunks)
                def _(u=u):
                    pltpu.make_async_copy(
                        feat_hbm.at[si.at[pp, u]], rows.at[u],
                        gsems[u]).wait()
                    for g in range(C // LANES):
                        sl = pl.ds(g * LANES, LANES)
                        svr = si[pp, u, sl] - off
                        dv = di[pp, u, sl]
                        gdv = plsc.load_gather(gd_t, [dv])
                        gsv = plsc.load_gather(gs_t, [svr])
                        z = gdv + gsv
                        t = jnp.exp(-2.0 * jnp.abs(z))
                        ev = jnp.sign(z) * (1.0 - t) / (1.0 + t)
                        for i in range(LANES):
                            s = ev[i]
                            r = g * LANES + i
                            for j in range(D // LANES):
                                fsl = pl.ds(j * LANES, LANES)
                                rows[u, r, fsl] = rows[u, r, fsl] * s
                    pltpu.async_copy(
                        rows.at[u], agg_sp.at[di.at[pp, u]], ssems[u],
                        add=True)
            return ()
        lax.fori_loop(0, nquads, _quad, ())
        for u in range(4):
            if 4 * (nquads - 1) + u < nchunks:
                _wait_scat(u)
        plsc.subcore_barrier()

        @pl.when(sid < nio)
        def _():
            pltpu.sync_copy(
                agg_sp.at[pl.ds(sid * rows_per_io, rows_per_io)],
                out_hbm.at[pl.ds(off + sid * rows_per_io, rows_per_io)])

    return msg_kernel


def _prep0_body(x_ref, cnt_ref, w_ref, b_ref, feat_ref, gg_ref, nm_ref):
    x = x_ref[...]
    R, D = x.shape
    deg = cnt_ref[0] + cnt_ref[1]
    nrm = lax.rsqrt(jnp.clip(deg, 1.0, None))
    nm_ref[...] = nrm
    xs = (x.reshape(R // 128, 128, D) * nrm[:, :, None]).reshape(R, D)
    feat_ref[0] = xs
    feat_ref[1] = xs
    w = w_ref[...]
    da = lax.dot_general(w[0:2], x, (((1,), (1,)), ((), ())),
                         preferred_element_type=_F32)
    db = lax.dot_general(w[2:4], x, (((1,), (1,)), ((), ())),
                         preferred_element_type=_F32)
    gg = jnp.concatenate([da, db], axis=0) + b_ref[0:4, 0:1]
    gg_ref[...] = gg.reshape(4, 8, 128)


def _update_body(agg_ref, x_ref, w_ref, b_ref, nm_ref, feat_ref, gg_ref):
    x = x_ref[...]
    R, D = x.shape
    fa = agg_ref[0] * (1.0 - BETA) + x * BETA
    fb = agg_ref[1] * (1.0 - BETA) + x * BETA
    nrm = nm_ref[...]
    feat_ref[0] = (fa.reshape(R // 128, 128, D) * nrm[:, :, None]).reshape(R, D)
    feat_ref[1] = (fb.reshape(R // 128, 128, D) * nrm[:, :, None]).reshape(R, D)
    w = w_ref[...]
    da = lax.dot_general(w[0:2], fa, (((1,), (1,)), ((), ())),
                         preferred_element_type=_F32)
    db = lax.dot_general(w[2:4], fb, (((1,), (1,)), ((), ())),
                         preferred_element_type=_F32)
    gg = jnp.concatenate([da, db], axis=0) + b_ref[0:4, 0:1]
    gg_ref[...] = gg.reshape(4, 8, 128)


def _final_body(agg_ref, x_ref, w_ref, b_ref, out_ref):
    x = x_ref[...]
    fa = agg_ref[0] * (1.0 - BETA) + x * BETA
    fb = agg_ref[1] * (1.0 - BETA) + x * BETA
    b = b_ref[...]
    oa = lax.dot_general(fa, w_ref[0], (((1,), (0,)), ((), ())),
                         preferred_element_type=_F32) + b[0:1, :]
    ob = lax.dot_general(fb, w_ref[1], (((1,), (0,)), ((), ())),
                         preferred_element_type=_F32) + b[1:2, :]
    D = oa.shape[1]
    out_ref[:, 0:D] = jnp.maximum(oa, 0.0)
    out_ref[:, D:2 * D] = jnp.maximum(ob, 0.0)


def kernel(x, edge_index, gate_W, gate_b, app_W, app_b):
    N, D = x.shape
    E = edge_index.shape[1]
    G = app_W.shape[0]
    H = gate_W.shape[0] // G
    grid = (pl.cdiv(N, _R),)
    NT = grid[0] * (_R // 128)
    src = edge_index[0].astype(_I32)
    dst = edge_index[1].astype(_I32)
    src2d = jnp.concatenate(
        [src.reshape(E // 80, 80), jnp.zeros((8, 80), _I32)], axis=0)
    dst2d = jnp.concatenate(
        [dst.reshape(E // 80, 80), jnp.zeros((8, 80), _I32)], axis=0)

    sc_deg = _make_sc_deg(N, E, NT)
    sc_msg = _make_sc_msg(N, D // 2, E, NT)

    def w8b8(h):
        ia, ib = h, H + h
        rows = jnp.stack([gate_W[ia, :D, 0], gate_W[ia, D:, 0],
                          gate_W[ib, :D, 0], gate_W[ib, D:, 0]])
        w8 = jnp.concatenate([rows, jnp.zeros((4, D), _F32)], axis=0)
        b8 = jnp.zeros((8, D), _F32)
        b8 = b8.at[0, 0].set(gate_b[ia, 0]).at[2, 0].set(gate_b[ib, 0])
        return w8, b8

    full = pl.BlockSpec((8, D), lambda i: (0, 0))

    cnt = sc_deg(dst)

    w8, b8 = w8b8(0)
    feat2, gdgs, norm = pl.pallas_call(
        _prep0_body,
        grid=grid,
        in_specs=[
            pl.BlockSpec((_R, D), lambda i: (i, 0)),
            pl.BlockSpec((2, 8, 128), lambda i: (0, i, 0)),
            full, full,
        ],
        out_specs=[
            pl.BlockSpec((2, _R, D), lambda i: (0, i, 0)),
            pl.BlockSpec((4, 8, 128), lambda i: (0, i, 0)),
            pl.BlockSpec((8, 128), lambda i: (i, 0)),
        ],
        out_shape=[
            jax.ShapeDtypeStruct((2, N, D), _F32),
            jax.ShapeDtypeStruct((4, NT, 128), _F32),
            jax.ShapeDtypeStruct((NT, 128), _F32),
        ],
    )(x, cnt, w8, b8)

    agg2 = None
    for h in range(H):
        if h > 0:
            w8, b8 = w8b8(h)
            feat2, gdgs = pl.pallas_call(
                _update_body,
                grid=grid,
                in_specs=[
                    pl.BlockSpec((2, _R, D), lambda i: (0, i, 0)),
                    pl.BlockSpec((_R, D), lambda i: (i, 0)),
                    full, full,
                    pl.BlockSpec((8, 128), lambda i: (i, 0)),
                ],
                out_specs=[
                    pl.BlockSpec((2, _R, D), lambda i: (0, i, 0)),
                    pl.BlockSpec((4, 8, 128), lambda i: (0, i, 0)),
                ],
                out_shape=[
                    jax.ShapeDtypeStruct((2, N, D), _F32),
                    jax.ShapeDtypeStruct((4, NT, 128), _F32),
                ],
            )(agg2, x, w8, b8, norm)
        f_flat = feat2.reshape(G * N, D)
        hw = D // 2
        agg_lo = sc_msg(f_flat[:, :hw], gdgs, src2d, dst2d)
        agg_hi = sc_msg(f_flat[:, hw:], gdgs, src2d, dst2d)
        agg2 = jnp.concatenate(
            [agg_lo.reshape(G, N, hw), agg_hi.reshape(G, N, hw)], axis=-1)

    ab8 = jnp.zeros((8, D), _F32).at[0:G].set(app_b)
    out = pl.pallas_call(
        _final_body,
        grid=grid,
        in_specs=[
            pl.BlockSpec((2, _R, D), lambda i: (0, i, 0)),
            pl.BlockSpec((_R, D), lambda i: (i, 0)),
            pl.BlockSpec((2, D, D), lambda i: (0, 0, 0)),
            full,
        ],
        out_specs=pl.BlockSpec((_R, 2 * D), lambda i: (i, 0)),
        out_shape=jax.ShapeDtypeStruct((N, 2 * D), _F32),
    )(agg2, x, app_W, ab8)
    return out


# speedup vs baseline: 1.2862x; 1.2862x over previous
import functools

import jax
import jax.numpy as jnp
from jax import lax
from jax.experimental import pallas as pl
from jax.experimental.pallas import tpu as pltpu
from jax.experimental.pallas import tpu_sc as plsc

BETA = 0.1
NC = 2
NS = 16
LANES = 16

_F32 = jnp.float32
_I32 = jnp.int32

_R = 1024


def _make_sc_deg(N, E, NT):
    C = 80
    W32 = NC * NS
    ep = E // W32
    nchunks = ep // C
    npad = (NT * 128) // NS
    mesh = plsc.VectorSubcoreMesh(
        core_axis_name="c", subcore_axis_name="s", num_cores=NC,
        num_subcores=NS)

    @functools.partial(
        pl.kernel, mesh=mesh,
        compiler_params=pltpu.CompilerParams(use_tc_tiling_on_sc=False, needs_layout_passes=False),
        out_type=jax.ShapeDtypeStruct((NC, NT, 128), _F32),
        scratch_types=[
            pltpu.VMEM((2, C), _I32),
            pltpu.VMEM((C,), _F32),
            pltpu.VMEM((1024,), _F32),
            pltpu.VMEM((8, 128), _F32),
            pltpu.VMEM_SHARED((NT * 128,), _F32),
            pltpu.SemaphoreType.DMA,
            pltpu.SemaphoreType.DMA,
        ],
    )
    def deg_kernel(dst_hbm, out_hbm, dst_row, ones, v1d, v2d, cnt_sp,
                   dsem0, dsem1):
        cid = lax.axis_index("c")
        sid = lax.axis_index("s")
        z16 = jnp.zeros((LANES,), _F32)
        o16 = jnp.ones((LANES,), _F32)
        dsems = (dsem0, dsem1)

        def _fill(i, _):
            v1d[pl.ds(i * LANES, LANES)] = z16
            return ()
        lax.fori_loop(0, 1024 // LANES, _fill, ())
        for i in range(C // LANES):
            ones[pl.ds(i * LANES, LANES)] = o16
        pltpu.sync_copy(v1d.at[pl.ds(0, npad)],
                        cnt_sp.at[pl.ds(sid * npad, npad)])
        plsc.subcore_barrier()

        base = (sid * NC + cid) * ep

        def _fire(c, b):
            pltpu.async_copy(
                dst_hbm.at[pl.ds(base + c * C, C)], dst_row.at[b], dsems[b])

        def _wait(b):
            pltpu.make_async_copy(
                dst_hbm.at[pl.ds(0, C)], dst_row.at[b], dsems[b]).wait()

        _fire(0, 0)

        def _chunk2(c2, _):
            for b in (0, 1):
                cc = 2 * c2 + b

                @pl.when(cc + 1 < nchunks)
                def _(b=b, cc=cc):
                    _fire(cc + 1, 1 - b)
                _wait(b)
                pltpu.sync_copy(ones, cnt_sp.at[dst_row.at[b]], add=True)
            return ()
        lax.fori_loop(0, nchunks // 2, _chunk2, ())
        if nchunks % 2:
            _wait(0)
            pltpu.sync_copy(ones, cnt_sp.at[dst_row.at[0]], add=True)
        plsc.subcore_barrier()

        @pl.when(sid < NT // 8)
        def _():
            pltpu.sync_copy(cnt_sp.at[pl.ds(sid * 1024, 1024)], v1d)
            for r in range(8):
                for c in range(128 // LANES):
                    v2d[r, pl.ds(c * LANES, LANES)] = (
                        v1d[pl.ds((r * 8 + c) * LANES, LANES)])
            pltpu.sync_copy(v2d, out_hbm.at[cid, pl.ds(sid * 8, 8)])

    return deg_kernel


def _make_sc_msg(N, D, E, NT):
    C = 80
    ep = E // NS
    nchunks = ep // C
    nio = 10
    rows_per_io = N // nio
    zr = 40
    mesh = plsc.VectorSubcoreMesh(
        core_axis_name="c", subcore_axis_name="s", num_cores=NC,
        num_subcores=NS)

    npairs = nchunks // 2
    cpt = nchunks

    @functools.partial(
        pl.kernel, mesh=mesh,
        compiler_params=pltpu.CompilerParams(use_tc_tiling_on_sc=False, needs_layout_passes=False),
        out_type=jax.ShapeDtypeStruct((NC * N, D), _F32),
        scratch_types=[
            pltpu.VMEM((2, 2, C), _I32),
            pltpu.VMEM((2, 2, C), _I32),
            pltpu.VMEM((NT * 128,), _F32),
            pltpu.VMEM((NT * 128,), _F32),
            pltpu.VMEM((2, C, D), _F32),
            pltpu.VMEM((zr, D), _F32),
            pltpu.VMEM_SHARED((N, D), _F32),
            pltpu.SemaphoreType.DMA,
            pltpu.SemaphoreType.DMA,
            pltpu.SemaphoreType.DMA,
            pltpu.SemaphoreType.DMA,
            pltpu.SemaphoreType.DMA,
        ],
    )
    def msg_kernel(feat_hbm, gdgs_hbm, src_hbm, dst_hbm, out_hbm,
                   si, di, gd_t, gs_t, rows, zbuf, agg_sp,
                   isem, gsem0, gsem1, ssem0, ssem1):
        cid = lax.axis_index("c")
        sid = lax.axis_index("s")
        off = cid * N
        gsems = (gsem0, gsem1)
        ssems = (ssem0, ssem1)

        def _flatten(t1d):
            def _row(r, _):
                for c in range(128 // LANES):
                    t1d[pl.ds(r * 128 + c * LANES, LANES)] = (
                        rows[0, r, pl.ds(c * LANES, LANES)])
                return ()
            lax.fori_loop(0, NT, _row, ())

        pltpu.sync_copy(gdgs_hbm.at[2 * cid], rows.at[0])
        _flatten(gd_t)
        pltpu.sync_copy(gdgs_hbm.at[2 * cid + 1], rows.at[0])
        _flatten(gs_t)

        z16 = jnp.zeros((LANES,), _F32)

        def _fill(r, _):
            for j in range(D // LANES):
                zbuf[r, pl.ds(j * LANES, LANES)] = z16
            return ()
        lax.fori_loop(0, zr, _fill, ())

        @pl.when(sid < nio)
        def _():
            def _zero(q, _):
                pltpu.sync_copy(
                    zbuf, agg_sp.at[pl.ds(sid * rows_per_io + q * zr, zr)])
                return ()
            lax.fori_loop(0, rows_per_io // zr, _zero, ())
        plsc.subcore_barrier()

        base_row = sid * cpt

        def _fire_idx(pair_idx, parity):
            row = base_row + 2 * pair_idx
            pltpu.async_copy(src_hbm.at[pl.ds(row, 2)], si.at[parity], isem)
            pltpu.async_copy(dst_hbm.at[pl.ds(row, 2)], di.at[parity], isem)

        def _wait_idx():
            pltpu.make_async_copy(
                src_hbm.at[pl.ds(0, 2)], si.at[0], isem).wait()
            pltpu.make_async_copy(
                dst_hbm.at[pl.ds(0, 2)], di.at[0], isem).wait()

        def _wait_scat(b):
            pltpu.make_async_copy(
                rows.at[b], agg_sp.at[pl.ds(0, C)], ssems[b]).wait()

        _fire_idx(0, 0)

        def _pair(P, _):
            pp = P & 1
            qq = 1 - pp
            _wait_idx()
            for b in (0, 1):
                for g in range(C // LANES):
                    sl = pl.ds(g * LANES, LANES)
                    si[pp, b, sl] = si[pp, b, sl] + off
            for b in (0, 1):
                @pl.when(P > 0)
                def _(b=b):
                    _wait_scat(b)
                pltpu.async_copy(
                    feat_hbm.at[si.at[pp, b]], rows.at[b], gsems[b])
            @pl.when(P + 1 < npairs)
            def _():
                _fire_idx(P + 1, qq)
            for b in (0, 1):
                pltpu.make_async_copy(
                    feat_hbm.at[si.at[pp, b]], rows.at[b], gsems[b]).wait()
                for g in range(C // LANES):
                    sl = pl.ds(g * LANES, LANES)
                    svr = si[pp, b, sl] - off
                    dv = di[pp, b, sl]
                    gdv = plsc.load_gather(gd_t, [dv])
                    gsv = plsc.load_gather(gs_t, [svr])
                    z = gdv + gsv
                    t = jnp.exp(-2.0 * jnp.abs(z))
                    ev = jnp.sign(z) * (1.0 - t) / (1.0 + t)
                    for i in range(LANES):
                        s = ev[i]
                        r = g * LANES + i
                        for j in range(D // LANES):
                            fsl = pl.ds(j * LANES, LANES)
                            rows[b, r, fsl] = rows[b, r, fsl] * s
                pltpu.async_copy(
                    rows.at[b], agg_sp.at[di.at[pp, b]], ssems[b], add=True)
            return ()
        lax.fori_loop(0, npairs, _pair, ())
        _wait_scat(0)
        _wait_scat(1)
        plsc.subcore_barrier()

        @pl.when(sid < nio)
        def _():
            pltpu.sync_copy(
                agg_sp.at[pl.ds(sid * rows_per_io, rows_per_io)],
                out_hbm.at[pl.ds(off + sid * rows_per_io, rows_per_io)])

    return msg_kernel


def _prep0_body(x_ref, cnt_ref, w_ref, b_ref, feat_ref, gg_ref, nm_ref):
    x = x_ref[...]
    R, D = x.shape
    deg = cnt_ref[0] + cnt_ref[1]
    nrm = lax.rsqrt(jnp.clip(deg, 1.0, None))
    nm_ref[...] = nrm
    xs = (x.reshape(R // 128, 128, D) * nrm[:, :, None]).reshape(R, D)
    feat_ref[0] = xs
    feat_ref[1] = xs
    w = w_ref[...]
    da = lax.dot_general(w[0:2], x, (((1,), (1,)), ((), ())),
                         preferred_element_type=_F32)
    db = lax.dot_general(w[2:4], x, (((1,), (1,)), ((), ())),
                         preferred_element_type=_F32)
    gg = jnp.concatenate([da, db], axis=0) + b_ref[0:4, 0:1]
    gg_ref[...] = gg.reshape(4, 8, 128)


def _update_body(agg_ref, x_ref, w_ref, b_ref, nm_ref, feat_ref, gg_ref):
    x = x_ref[...]
    R, D = x.shape
    fa = agg_ref[0] * (1.0 - BETA) + x * BETA
    fb = agg_ref[1] * (1.0 - BETA) + x * BETA
    nrm = nm_ref[...]
    feat_ref[0] = (fa.reshape(R // 128, 128, D) * nrm[:, :, None]).reshape(R, D)
    feat_ref[1] = (fb.reshape(R // 128, 128, D) * nrm[:, :, None]).reshape(R, D)
    w = w_ref[...]
    da = lax.dot_general(w[0:2], fa, (((1,), (1,)), ((), ())),
                         preferred_element_type=_F32)
    db = lax.dot_general(w[2:4], fb, (((1,), (1,)), ((), ())),
                         preferred_element_type=_F32)
    gg = jnp.concatenate([da, db], axis=0) + b_ref[0:4, 0:1]
    gg_ref[...] = gg.reshape(4, 8, 128)


def _final_body(agg_ref, x_ref, w_ref, b_ref, out_ref):
    x = x_ref[...]
    fa = agg_ref[0] * (1.0 - BETA) + x * BETA
    fb = agg_ref[1] * (1.0 - BETA) + x * BETA
    b = b_ref[...]
    oa = lax.dot_general(fa, w_ref[0], (((1,), (0,)), ((), ())),
                         preferred_element_type=_F32) + b[0:1, :]
    ob = lax.dot_general(fb, w_ref[1], (((1,), (0,)), ((), ())),
                         preferred_element_type=_F32) + b[1:2, :]
    D = oa.shape[1]
    out_ref[:, 0:D] = jnp.maximum(oa, 0.0)
    out_ref[:, D:2 * D] = jnp.maximum(ob, 0.0)


def kernel(x, edge_index, gate_W, gate_b, app_W, app_b):
    N, D = x.shape
    E = edge_index.shape[1]
    G = app_W.shape[0]
    H = gate_W.shape[0] // G
    grid = (pl.cdiv(N, _R),)
    NT = grid[0] * (_R // 128)
    src = edge_index[0].astype(_I32)
    dst = edge_index[1].astype(_I32)
    src2d = src.reshape(E // 80, 80)
    dst2d = dst.reshape(E // 80, 80)

    sc_deg = _make_sc_deg(N, E, NT)
    sc_msg = _make_sc_msg(N, D, E, NT)

    def w8b8(h):
        ia, ib = h, H + h
        rows = jnp.stack([gate_W[ia, :D, 0], gate_W[ia, D:, 0],
                          gate_W[ib, :D, 0], gate_W[ib, D:, 0]])
        w8 = jnp.concatenate([rows, jnp.zeros((4, D), _F32)], axis=0)
        b8 = jnp.zeros((8, D), _F32)
        b8 = b8.at[0, 0].set(gate_b[ia, 0]).at[2, 0].set(gate_b[ib, 0])
        return w8, b8

    full = pl.BlockSpec((8, D), lambda i: (0, 0))

    cnt = sc_deg(dst)

    w8, b8 = w8b8(0)
    feat2, gdgs, norm = pl.pallas_call(
        _prep0_body,
        grid=grid,
        in_specs=[
            pl.BlockSpec((_R, D), lambda i: (i, 0)),
            pl.BlockSpec((2, 8, 128), lambda i: (0, i, 0)),
            full, full,
        ],
        out_specs=[
            pl.BlockSpec((2, _R, D), lambda i: (0, i, 0)),
            pl.BlockSpec((4, 8, 128), lambda i: (0, i, 0)),
            pl.BlockSpec((8, 128), lambda i: (i, 0)),
        ],
        out_shape=[
            jax.ShapeDtypeStruct((2, N, D), _F32),
            jax.ShapeDtypeStruct((4, NT, 128), _F32),
            jax.ShapeDtypeStruct((NT, 128), _F32),
        ],
    )(x, cnt, w8, b8)

    agg2 = None
    for h in range(H):
        if h > 0:
            w8, b8 = w8b8(h)
            feat2, gdgs = pl.pallas_call(
                _update_body,
                grid=grid,
                in_specs=[
                    pl.BlockSpec((2, _R, D), lambda i: (0, i, 0)),
                    pl.BlockSpec((_R, D), lambda i: (i, 0)),
                    full, full,
                    pl.BlockSpec((8, 128), lambda i: (i, 0)),
                ],
                out_specs=[
                    pl.BlockSpec((2, _R, D), lambda i: (0, i, 0)),
                    pl.BlockSpec((4, 8, 128), lambda i: (0, i, 0)),
                ],
                out_shape=[
                    jax.ShapeDtypeStruct((2, N, D), _F32),
                    jax.ShapeDtypeStruct((4, NT, 128), _F32),
                ],
            )(agg2, x, w8, b8, norm)
        agg_flat = sc_msg(feat2.reshape(G * N, D), gdgs, src2d, dst2d)
        agg2 = agg_flat.reshape(G, N, D)

    ab8 = jnp.zeros((8, D), _F32).at[0:G].set(app_b)
    out = pl.pallas_call(
        _final_body,
        grid=grid,
        in_specs=[
            pl.BlockSpec((2, _R, D), lambda i: (0, i, 0)),
            pl.BlockSpec((_R, D), lambda i: (i, 0)),
            pl.BlockSpec((2, D, D), lambda i: (0, 0, 0)),
            full,
        ],
        out_specs=pl.BlockSpec((_R, 2 * D), lambda i: (i, 0)),
        out_shape=jax.ShapeDtypeStruct((N, 2 * D), _F32),
    )(agg2, x, app_W, ab8)
    return out


# speedup vs baseline: 1.3295x; 1.0337x over previous
import functools

import jax
import jax.numpy as jnp
from jax import lax
from jax.experimental import pallas as pl
from jax.experimental.pallas import tpu as pltpu
from jax.experimental.pallas import tpu_sc as plsc

BETA = 0.1
NC = 2
NS = 16
LANES = 16

_F32 = jnp.float32
_I32 = jnp.int32

_R = 1024


def _make_sc_deg(N, E, NT):
    C = 80
    W32 = NC * NS
    ep = E // W32
    nchunks = ep // C
    npad = (NT * 128) // NS
    mesh = plsc.VectorSubcoreMesh(
        core_axis_name="c", subcore_axis_name="s", num_cores=NC,
        num_subcores=NS)

    @functools.partial(
        pl.kernel, mesh=mesh,
        compiler_params=pltpu.CompilerParams(use_tc_tiling_on_sc=False, needs_layout_passes=False),
        out_type=jax.ShapeDtypeStruct((NC, NT, 128), _F32),
        scratch_types=[
            pltpu.VMEM((2, C), _I32),
            pltpu.VMEM((C,), _F32),
            pltpu.VMEM((1024,), _F32),
            pltpu.VMEM((8, 128), _F32),
            pltpu.VMEM_SHARED((NT * 128,), _F32),
            pltpu.SemaphoreType.DMA,
            pltpu.SemaphoreType.DMA,
        ],
    )
    def deg_kernel(dst_hbm, out_hbm, dst_row, ones, v1d, v2d, cnt_sp,
                   dsem0, dsem1):
        cid = lax.axis_index("c")
        sid = lax.axis_index("s")
        z16 = jnp.zeros((LANES,), _F32)
        o16 = jnp.ones((LANES,), _F32)
        dsems = (dsem0, dsem1)

        def _fill(i, _):
            v1d[pl.ds(i * LANES, LANES)] = z16
            return ()
        lax.fori_loop(0, 1024 // LANES, _fill, ())
        for i in range(C // LANES):
            ones[pl.ds(i * LANES, LANES)] = o16
        pltpu.sync_copy(v1d.at[pl.ds(0, npad)],
                        cnt_sp.at[pl.ds(sid * npad, npad)])
        plsc.subcore_barrier()

        base = (sid * NC + cid) * ep

        def _fire(c, b):
            pltpu.async_copy(
                dst_hbm.at[pl.ds(base + c * C, C)], dst_row.at[b], dsems[b])

        def _wait(b):
            pltpu.make_async_copy(
                dst_hbm.at[pl.ds(0, C)], dst_row.at[b], dsems[b]).wait()

        _fire(0, 0)

        def _chunk2(c2, _):
            for b in (0, 1):
                cc = 2 * c2 + b

                @pl.when(cc + 1 < nchunks)
                def _(b=b, cc=cc):
                    _fire(cc + 1, 1 - b)
                _wait(b)
                pltpu.sync_copy(ones, cnt_sp.at[dst_row.at[b]], add=True)
            return ()
        lax.fori_loop(0, nchunks // 2, _chunk2, ())
        if nchunks % 2:
            _wait(0)
            pltpu.sync_copy(ones, cnt_sp.at[dst_row.at[0]], add=True)
        plsc.subcore_barrier()

        @pl.when(sid < NT // 8)
        def _():
            pltpu.sync_copy(cnt_sp.at[pl.ds(sid * 1024, 1024)], v1d)
            for r in range(8):
                for c in range(128 // LANES):
                    v2d[r, pl.ds(c * LANES, LANES)] = (
                        v1d[pl.ds((r * 8 + c) * LANES, LANES)])
            pltpu.sync_copy(v2d, out_hbm.at[cid, pl.ds(sid * 8, 8)])

    return deg_kernel


def _make_sc_msg(N, D, E, NT):
    C = 80
    ep = E // NS
    nchunks = ep // C
    nio = 10
    rows_per_io = N // nio
    zr = 40
    mesh = plsc.VectorSubcoreMesh(
        core_axis_name="c", subcore_axis_name="s", num_cores=NC,
        num_subcores=NS)

    npairs = nchunks // 2
    cpt = nchunks

    @functools.partial(
        pl.kernel, mesh=mesh,
        compiler_params=pltpu.CompilerParams(use_tc_tiling_on_sc=False, needs_layout_passes=False),
        out_type=jax.ShapeDtypeStruct((NC * N, D), _F32),
        scratch_types=[
            pltpu.VMEM((2, 2, C), _I32),
            pltpu.VMEM((2, 2, C), _I32),
            pltpu.VMEM((NT * 128,), _F32),
            pltpu.VMEM((NT * 128,), _F32),
            pltpu.VMEM((2, C, D), _F32),
            pltpu.VMEM((zr, D), _F32),
            pltpu.VMEM_SHARED((N, D), _F32),
            pltpu.SemaphoreType.DMA,
            pltpu.SemaphoreType.DMA,
            pltpu.SemaphoreType.DMA,
            pltpu.SemaphoreType.DMA,
            pltpu.SemaphoreType.DMA,
        ],
    )
    def msg_kernel(feat_hbm, gdgs_hbm, src_hbm, dst_hbm, out_hbm,
                   si, di, gd_t, gs_t, rows, zbuf, agg_sp,
                   isem, gsem0, gsem1, ssem0, ssem1):
        cid = lax.axis_index("c")
        sid = lax.axis_index("s")
        off = cid * N
        gsems = (gsem0, gsem1)
        ssems = (ssem0, ssem1)

        def _flatten(t1d):
            def _row(r, _):
                for c in range(128 // LANES):
                    t1d[pl.ds(r * 128 + c * LANES, LANES)] = (
                        rows[0, r, pl.ds(c * LANES, LANES)])
                return ()
            lax.fori_loop(0, NT, _row, ())

        pltpu.sync_copy(gdgs_hbm.at[2 * cid], rows.at[0])
        _flatten(gd_t)
        pltpu.sync_copy(gdgs_hbm.at[2 * cid + 1], rows.at[0])
        _flatten(gs_t)

        z16 = jnp.zeros((LANES,), _F32)

        def _fill(r, _):
            for j in range(D // LANES):
                zbuf[r, pl.ds(j * LANES, LANES)] = z16
            return ()
        lax.fori_loop(0, zr, _fill, ())

        @pl.when(sid < nio)
        def _():
            def _zero(q, _):
                pltpu.sync_copy(
                    zbuf, agg_sp.at[pl.ds(sid * rows_per_io + q * zr, zr)])
                return ()
            lax.fori_loop(0, rows_per_io // zr, _zero, ())
        plsc.subcore_barrier()

        base_row = sid * cpt

        def _fire_idx(pair_idx, parity):
            row = base_row + 2 * pair_idx
            pltpu.async_copy(src_hbm.at[pl.ds(row, 2)], si.at[parity], isem)
            pltpu.async_copy(dst_hbm.at[pl.ds(row, 2)], di.at[parity], isem)

        def _wait_idx():
            pltpu.make_async_copy(
                src_hbm.at[pl.ds(0, 2)], si.at[0], isem).wait()
            pltpu.make_async_copy(
                dst_hbm.at[pl.ds(0, 2)], di.at[0], isem).wait()

        def _wait_scat(b):
            pltpu.make_async_copy(
                rows.at[b], agg_sp.at[pl.ds(0, C)], ssems[b]).wait()

        def _offset_add(parity):
            for b in (0, 1):
                for g in range(C // LANES):
                    sl = pl.ds(g * LANES, LANES)
                    si[parity, b, sl] = si[parity, b, sl] + off

        _fire_idx(0, 0)
        _wait_idx()
        _offset_add(0)
        pltpu.async_copy(feat_hbm.at[si.at[0, 0]], rows.at[0], gsems[0])

        def _pair(P, _):
            pp = P & 1
            qq = 1 - pp
            @pl.when(P > 0)
            def _():
                _wait_scat(1)
            pltpu.async_copy(
                feat_hbm.at[si.at[pp, 1]], rows.at[1], gsems[1])
            @pl.when(P + 1 < npairs)
            def _():
                _fire_idx(P + 1, qq)
            for b in (0, 1):
                pltpu.make_async_copy(
                    feat_hbm.at[si.at[pp, b]], rows.at[b], gsems[b]).wait()
                for g in range(C // LANES):
                    sl = pl.ds(g * LANES, LANES)
                    svr = si[pp, b, sl] - off
                    dv = di[pp, b, sl]
                    gdv = plsc.load_gather(gd_t, [dv])
                    gsv = plsc.load_gather(gs_t, [svr])
                    z = gdv + gsv
                    t = jnp.exp(-2.0 * jnp.abs(z))
                    ev = jnp.sign(z) * (1.0 - t) / (1.0 + t)
                    for i in range(LANES):
                        s = ev[i]
                        r = g * LANES + i
                        for j in range(D // LANES):
                            fsl = pl.ds(j * LANES, LANES)
                            rows[b, r, fsl] = rows[b, r, fsl] * s
                pltpu.async_copy(
                    rows.at[b], agg_sp.at[di.at[pp, b]], ssems[b], add=True)
            @pl.when(P + 1 < npairs)
            def _():
                _wait_idx()
                _offset_add(qq)
                _wait_scat(0)
                pltpu.async_copy(
                    feat_hbm.at[si.at[qq, 0]], rows.at[0], gsems[0])
            return ()
        lax.fori_loop(0, npairs, _pair, ())
        _wait_scat(0)
        _wait_scat(1)
        plsc.subcore_barrier()

        @pl.when(sid < nio)
        def _():
            pltpu.sync_copy(
                agg_sp.at[pl.ds(sid * rows_per_io, rows_per_io)],
                out_hbm.at[pl.ds(off + sid * rows_per_io, rows_per_io)])

    return msg_kernel


def _prep0_body(x_ref, cnt_ref, w_ref, b_ref, feat_ref, gg_ref, nm_ref):
    x = x_ref[...]
    R, D = x.shape
    deg = cnt_ref[0] + cnt_ref[1]
    nrm = lax.rsqrt(jnp.clip(deg, 1.0, None))
    nm_ref[...] = nrm
    xs = (x.reshape(R // 128, 128, D) * nrm[:, :, None]).reshape(R, D)
    feat_ref[0] = xs
    feat_ref[1] = xs
    w = w_ref[...]
    da = lax.dot_general(w[0:2], x, (((1,), (1,)), ((), ())),
                         preferred_element_type=_F32)
    db = lax.dot_general(w[2:4], x, (((1,), (1,)), ((), ())),
                         preferred_element_type=_F32)
    gg = jnp.concatenate([da, db], axis=0) + b_ref[0:4, 0:1]
    gg_ref[...] = gg.reshape(4, 8, 128)


def _update_body(agg_ref, x_ref, w_ref, b_ref, nm_ref, feat_ref, gg_ref):
    x = x_ref[...]
    R, D = x.shape
    fa = agg_ref[0] * (1.0 - BETA) + x * BETA
    fb = agg_ref[1] * (1.0 - BETA) + x * BETA
    nrm = nm_ref[...]
    feat_ref[0] = (fa.reshape(R // 128, 128, D) * nrm[:, :, None]).reshape(R, D)
    feat_ref[1] = (fb.reshape(R // 128, 128, D) * nrm[:, :, None]).reshape(R, D)
    w = w_ref[...]
    da = lax.dot_general(w[0:2], fa, (((1,), (1,)), ((), ())),
                         preferred_element_type=_F32)
    db = lax.dot_general(w[2:4], fb, (((1,), (1,)), ((), ())),
                         preferred_element_type=_F32)
    gg = jnp.concatenate([da, db], axis=0) + b_ref[0:4, 0:1]
    gg_ref[...] = gg.reshape(4, 8, 128)


def _final_body(agg_ref, x_ref, w_ref, b_ref, out_ref):
    x = x_ref[...]
    fa = agg_ref[0] * (1.0 - BETA) + x * BETA
    fb = agg_ref[1] * (1.0 - BETA) + x * BETA
    b = b_ref[...]
    oa = lax.dot_general(fa, w_ref[0], (((1,), (0,)), ((), ())),
                         preferred_element_type=_F32) + b[0:1, :]
    ob = lax.dot_general(fb, w_ref[1], (((1,), (0,)), ((), ())),
                         preferred_element_type=_F32) + b[1:2, :]
    D = oa.shape[1]
    out_ref[:, 0:D] = jnp.maximum(oa, 0.0)
    out_ref[:, D:2 * D] = jnp.maximum(ob, 0.0)


def kernel(x, edge_index, gate_W, gate_b, app_W, app_b):
    N, D = x.shape
    E = edge_index.shape[1]
    G = app_W.shape[0]
    H = gate_W.shape[0] // G
    grid = (pl.cdiv(N, _R),)
    NT = grid[0] * (_R // 128)
    src = edge_index[0].astype(_I32)
    dst = edge_index[1].astype(_I32)
    src2d = src.reshape(E // 80, 80)
    dst2d = dst.reshape(E // 80, 80)

    sc_deg = _make_sc_deg(N, E, NT)
    sc_msg = _make_sc_msg(N, D, E, NT)

    def w8b8(h):
        ia, ib = h, H + h
        rows = jnp.stack([gate_W[ia, :D, 0], gate_W[ia, D:, 0],
                          gate_W[ib, :D, 0], gate_W[ib, D:, 0]])
        w8 = jnp.concatenate([rows, jnp.zeros((4, D), _F32)], axis=0)
        b8 = jnp.zeros((8, D), _F32)
        b8 = b8.at[0, 0].set(gate_b[ia, 0]).at[2, 0].set(gate_b[ib, 0])
        return w8, b8

    full = pl.BlockSpec((8, D), lambda i: (0, 0))

    cnt = sc_deg(dst)

    w8, b8 = w8b8(0)
    feat2, gdgs, norm = pl.pallas_call(
        _prep0_body,
        grid=grid,
        in_specs=[
            pl.BlockSpec((_R, D), lambda i: (i, 0)),
            pl.BlockSpec((2, 8, 128), lambda i: (0, i, 0)),
            full, full,
        ],
        out_specs=[
            pl.BlockSpec((2, _R, D), lambda i: (0, i, 0)),
            pl.BlockSpec((4, 8, 128), lambda i: (0, i, 0)),
            pl.BlockSpec((8, 128), lambda i: (i, 0)),
        ],
        out_shape=[
            jax.ShapeDtypeStruct((2, N, D), _F32),
            jax.ShapeDtypeStruct((4, NT, 128), _F32),
            jax.ShapeDtypeStruct((NT, 128), _F32),
        ],
    )(x, cnt, w8, b8)

    agg2 = None
    for h in range(H):
        if h > 0:
            w8, b8 = w8b8(h)
            feat2, gdgs = pl.pallas_call(
                _update_body,
                grid=grid,
                in_specs=[
                    pl.BlockSpec((2, _R, D), lambda i: (0, i, 0)),
                    pl.BlockSpec((_R, D), lambda i: (i, 0)),
                    full, full,
                    pl.BlockSpec((8, 128), lambda i: (i, 0)),
                ],
                out_specs=[
                    pl.BlockSpec((2, _R, D), lambda i: (0, i, 0)),
                    pl.BlockSpec((4, 8, 128), lambda i: (0, i, 0)),
                ],
                out_shape=[
                    jax.ShapeDtypeStruct((2, N, D), _F32),
                    jax.ShapeDtypeStruct((4, NT, 128), _F32),
                ],
            )(agg2, x, w8, b8, norm)
        agg_flat = sc_msg(feat2.reshape(G * N, D), gdgs, src2d, dst2d)
        agg2 = agg_flat.reshape(G, N, D)

    ab8 = jnp.zeros((8, D), _F32).at[0:G].set(app_b)
    out = pl.pallas_call(
        _final_body,
        grid=grid,
        in_specs=[
            pl.BlockSpec((2, _R, D), lambda i: (0, i, 0)),
            pl.BlockSpec((_R, D), lambda i: (i, 0)),
            pl.BlockSpec((2, D, D), lambda i: (0, 0, 0)),
            full,
        ],
        out_specs=pl.BlockSpec((_R, 2 * D), lambda i: (i, 0)),
        out_shape=jax.ShapeDtypeStruct((N, 2 * D), _F32),
    )(agg2, x, app_W, ab8)
    return out
